# baseline (speedup 1.0000x reference)
import functools

import jax
import jax.numpy as jnp
from jax import lax
from jax.experimental import pallas as pl
from jax.experimental.pallas import tpu as pltpu
from jax.experimental.pallas import tpu_sc as plsc

N = 10000
E = 320000
D = 128
P = 8192
D2 = D // 2

NC = 2
NS = 16
NW = NC * NS

NP = 10240
RPT = NP // NS
HR = NP // 16
RPT16 = HR // NS
IDC = HR // 128

EPW = E // NW
K = 80
NCH = EPW // K

PPW = 4 * P // NW
PK = 128
PCH = PPW // PK

_mesh = plsc.VectorSubcoreMesh(core_axis_name="c", subcore_axis_name="s")


def _wid():
    return lax.axis_index("s") * NC + lax.axis_index("c")


HB = NP // 128
HBT = 8


@functools.partial(
    pl.kernel,
    out_type=jax.ShapeDtypeStruct((NC, 2, HB, 128), jnp.float32),
    mesh=_mesh,
    compiler_params=pltpu.CompilerParams(needs_layout_passes=False,
                                         use_tc_tiling_on_sc=False),
    scratch_types=[
        pltpu.VMEM((EPW,), jnp.int32),
        pltpu.VMEM((EPW,), jnp.int32),
        pltpu.VMEM((NP,), jnp.float32),
        pltpu.VMEM((NP,), jnp.float32),
        pltpu.VMEM((HB, 128), jnp.float32),
        pltpu.VMEM((HB, 128), jnp.float32),
        pltpu.VMEM((HB,), jnp.int32),
        pltpu.VMEM((HBT, 128), jnp.float32),
        pltpu.VMEM_SHARED((HB, 128), jnp.float32),
        pltpu.VMEM_SHARED((HB, 128), jnp.float32),
    ],
)
def _sc_degrees(s_hbm, r_hbm, out_hbm, sidx_v, ridx_v, hist_s1, hist_r1,
                hist_s, hist_r, idv, zb, cnt_s_sh, cnt_r_sh):
    cid = lax.axis_index("c")
    sid = lax.axis_index("s")
    base = _wid() * EPW
    pltpu.sync_copy(s_hbm.at[pl.ds(base, EPW)], sidx_v)
    pltpu.sync_copy(r_hbm.at[pl.ds(base, EPW)], ridx_v)

    zeros16 = jnp.zeros((16,), jnp.float32)
    ones16 = jnp.ones((16,), jnp.float32)
    iota16 = lax.iota(jnp.int32, 16)

    def _zero_hist(i, _):
        hist_s1[pl.ds(i * 16, 16)] = zeros16
        hist_r1[pl.ds(i * 16, 16)] = zeros16
        return 0

    lax.fori_loop(0, NP // 16, _zero_hist, 0)
    for i in range(HBT):
        for j in range(8):
            zb[i, pl.ds(j * 16, 16)] = zeros16
    for g in range(HB // 16):
        idv[pl.ds(g * 16, 16)] = g * 16 + iota16

    @pl.when(sid < HB // HBT)
    def _zero_counts():
        pltpu.sync_copy(zb, cnt_s_sh.at[pl.ds(sid * HBT, HBT)])
        pltpu.sync_copy(zb, cnt_r_sh.at[pl.ds(sid * HBT, HBT)])
    plsc.subcore_barrier()

    def _hist(i, _):
        sv = sidx_v[pl.ds(i * 16, 16)]
        plsc.addupdate_scatter(hist_s1, [sv], ones16)
        rv = ridx_v[pl.ds(i * 16, 16)]
        plsc.addupdate_scatter(hist_r1, [rv], ones16)
        return 0

    lax.fori_loop(0, EPW // 16, _hist, 0)

    def _pack(i, _):
        for j in range(8):
            hist_s[i, pl.ds(j * 16, 16)] = hist_s1[pl.ds(i * 128 + j * 16, 16)]
            hist_r[i, pl.ds(j * 16, 16)] = hist_r1[pl.ds(i * 128 + j * 16, 16)]
        return 0

    lax.fori_loop(0, HB, _pack, 0)

    pltpu.sync_copy(hist_s, cnt_s_sh.at[idv], add=True)
    pltpu.sync_copy(hist_r, cnt_r_sh.at[idv], add=True)
    plsc.subcore_barrier()

    @pl.when(sid < HB // HBT)
    def _write_counts():
        pltpu.sync_copy(cnt_s_sh.at[pl.ds(sid * HBT, HBT)],
                        out_hbm.at[cid, 0, pl.ds(sid * HBT, HBT)])
        pltpu.sync_copy(cnt_r_sh.at[pl.ds(sid * HBT, HBT)],
                        out_hbm.at[cid, 1, pl.ds(sid * HBT, HBT)])


@functools.partial(
    pl.kernel,
    out_type=jax.ShapeDtypeStruct((NC, NP, D), jnp.float32),
    mesh=_mesh,
    compiler_params=pltpu.CompilerParams(needs_layout_passes=False,
                                         use_tc_tiling_on_sc=True),
    scratch_types=[
        pltpu.VMEM((K,), jnp.int32), pltpu.VMEM((K,), jnp.int32),
        pltpu.VMEM((K,), jnp.int32), pltpu.VMEM((K,), jnp.int32),
        pltpu.VMEM((2, K, D), jnp.float32),
        pltpu.VMEM_SHARED((NP, D), jnp.float32),
        pltpu.SemaphoreType.DMA, pltpu.SemaphoreType.DMA,
        pltpu.SemaphoreType.DMA, pltpu.SemaphoreType.DMA,
    ],
)
def _sc_segsum(xs_hbm, sidx_hbm, ridx_hbm, out_hbm,
               sc0, sc1, rc0, rc1, rows_v, acc_sh, si0, si1, sg0, sg1):
    cid = lax.axis_index("c")
    sid = lax.axis_index("s")
    base = _wid() * EPW

    scurs = (sc0, sc1)
    rcurs = (rc0, rc1)
    sems_i = (si0, si1)
    sems_g = (sg0, sg1)

    zeros16 = jnp.zeros((16,), jnp.float32)

    def _zero_row(i, _):
        for j in range(D // 16):
            rows_v[0, i, pl.ds(j * 16, 16)] = zeros16
        return 0

    lax.fori_loop(0, K, _zero_row, 0)
    for r in range(RPT // K):
        pltpu.async_copy(rows_v.at[0],
                         acc_sh.at[pl.ds(sid * RPT + r * K, K)], si0)
    for r in range(RPT // K):
        pltpu.make_async_copy(rows_v.at[0],
                              acc_sh.at[pl.ds(sid * RPT + r * K, K)],
                              si0).wait()
    plsc.subcore_barrier()

    def _fire_idx(cc, j):
        pltpu.async_copy(sidx_hbm.at[pl.ds(base + cc * K, K)], scurs[j],
                         sems_i[j])
        pltpu.async_copy(ridx_hbm.at[pl.ds(base + cc * K, K)], rcurs[j],
                         sems_i[j])

    def _wait_idx(cc, j):
        pltpu.make_async_copy(sidx_hbm.at[pl.ds(base + cc * K, K)], scurs[j],
                              sems_i[j]).wait()
        pltpu.make_async_copy(ridx_hbm.at[pl.ds(base + cc * K, K)], rcurs[j],
                              sems_i[j]).wait()

    def _fire_g(j, b):
        pltpu.async_copy(xs_hbm.at[scurs[j]], rows_v.at[b], sems_g[b])

    def _drain_scatter(j, b):
        pltpu.make_async_copy(xs_hbm.at[scurs[j]], rows_v.at[b],
                              sems_g[b]).wait()
        pltpu.sync_copy(rows_v.at[b], acc_sh.at[rcurs[j]], add=True)

    _fire_idx(0, 0)
    _fire_idx(1, 1)
    _wait_idx(0, 0)
    _fire_g(0, 0)

    def _body(hc, _):
        c0 = hc * 2
        _wait_idx(c0 + 1, 1)
        _fire_g(1, 1)
        _drain_scatter(0, 0)
        _fire_idx(c0 + 2, 0)
        _drain_scatter(1, 1)

        @pl.when(c0 + 3 < NCH)
        def _():
            _fire_idx(c0 + 3, 1)
        _wait_idx(c0 + 2, 0)
        _fire_g(0, 0)
        return 0

    lax.fori_loop(0, (NCH - 1) // 2, _body, 0)
    _drain_scatter(0, 0)

    plsc.subcore_barrier()
    semo = (si0, si1)
    for r in range(RPT // K):
        b = r % 2
        if r >= 2:
            pltpu.make_async_copy(
                rows_v.at[b],
                out_hbm.at[cid].at[pl.ds(sid * RPT + (r - 2) * K, K)],
                semo[b]).wait()
        pltpu.sync_copy(acc_sh.at[pl.ds(sid * RPT + r * K, K)], rows_v.at[b])
        pltpu.async_copy(rows_v.at[b],
                         out_hbm.at[cid].at[pl.ds(sid * RPT + r * K, K)],
                         semo[b])
    for r in range(RPT // K - 2, RPT // K):
        b = r % 2
        pltpu.make_async_copy(
            rows_v.at[b],
            out_hbm.at[cid].at[pl.ds(sid * RPT + r * K, K)], semo[b]).wait()


@functools.partial(
    pl.kernel,
    out_type=jax.ShapeDtypeStruct((4 * P, D), jnp.float32),
    mesh=_mesh,
    compiler_params=pltpu.CompilerParams(needs_layout_passes=False,
                                         use_tc_tiling_on_sc=True),
    scratch_types=[
        pltpu.VMEM((PPW,), jnp.int32),
        pltpu.VMEM((2, PK, D), jnp.float32),
        pltpu.SemaphoreType.DMA,
        pltpu.SemaphoreType.DMA,
    ],
)
def _sc_pair_gather(h_hbm, idx_hbm, out_hbm, idx_v, rows_v, sem0, sem1):
    wid = _wid()
    pltpu.sync_copy(idx_hbm.at[wid], idx_v)
    sems = (sem0, sem1)

    def _fire(c, b):
        pltpu.async_copy(h_hbm.at[idx_v.at[pl.ds(c * PK, PK)]], rows_v.at[b],
                         sems[b])

    def _drain_write(c, b):
        pltpu.make_async_copy(h_hbm.at[idx_v.at[pl.ds(c * PK, PK)]],
                              rows_v.at[b], sems[b]).wait()
        pltpu.sync_copy(rows_v.at[b],
                        out_hbm.at[pl.ds(wid * PPW + c * PK, PK)])

    _fire(0, 0)
    for c in range(PCH):
        if c + 1 < PCH:
            _fire(c + 1, (c + 1) % 2)
        _drain_write(c, c % 2)


_BN = 1024


def _tc_scales(cnt, embed_p):
    def body(cnt_ref, emb_ref, ss_ref, rs_ref, xs_ref):
        cs = cnt_ref[0, 0] + cnt_ref[1, 0] + 1.0
        cr = cnt_ref[0, 1] + cnt_ref[1, 1] + 1.0
        ss = lax.rsqrt(cs)
        ss_ref[...] = ss
        rs_ref[...] = lax.rsqrt(cr) / cr
        xs_ref[...] = emb_ref[...] * ss

    return pl.pallas_call(
        body,
        grid=(NP // _BN,),
        in_specs=[
            pl.BlockSpec((NC, 2, _BN, 1), lambda i: (0, 0, i, 0)),
            pl.BlockSpec((_BN, D), lambda i: (i, 0)),
        ],
        out_specs=[
            pl.BlockSpec((_BN, 1), lambda i: (i, 0)),
            pl.BlockSpec((_BN, 1), lambda i: (i, 0)),
            pl.BlockSpec((_BN, D), lambda i: (i, 0)),
        ],
        out_shape=[
            jax.ShapeDtypeStruct((NP, 1), jnp.float32),
            jax.ShapeDtypeStruct((NP, 1), jnp.float32),
            jax.ShapeDtypeStruct((NP, D), jnp.float32),
        ],
    )(cnt, embed_p)


def _tc_layer(x, acc, xs, rs, ss, wt, wb, b, *, relu_and_next):
    def body(*refs):
        if relu_and_next:
            (x_ref, acc_ref, xs_ref, rs_ref, ss_ref, wt_ref, wb_ref, b_ref,
             h_ref, xs2_ref) = refs
        else:
            (x_ref, acc_ref, xs_ref, rs_ref, wt_ref, wb_ref, b_ref,
             h_ref) = refs
        m = (acc_ref[0] + acc_ref[1] + xs_ref[...]) * rs_ref[...]
        h = (jnp.dot(x_ref[...], wt_ref[...],
                     preferred_element_type=jnp.float32)
             + jnp.dot(m, wb_ref[...], preferred_element_type=jnp.float32)
             + b_ref[...])
        if relu_and_next:
            h = jnp.maximum(h, 0.0)
            h_ref[...] = h
            xs2_ref[...] = h * ss_ref[...]
        else:
            h_ref[...] = h

    in_specs = [
        pl.BlockSpec((_BN, D), lambda i: (i, 0)),
        pl.BlockSpec((NC, _BN, D), lambda i: (0, i, 0)),
        pl.BlockSpec((_BN, D), lambda i: (i, 0)),
        pl.BlockSpec((_BN, 1), lambda i: (i, 0)),
    ]
    args = [x, acc, xs, rs]
    if relu_and_next:
        in_specs.append(pl.BlockSpec((_BN, 1), lambda i: (i, 0)))
        args.append(ss)
    in_specs += [
        pl.BlockSpec((D, D), lambda i: (0, 0)),
        pl.BlockSpec((D, D), lambda i: (0, 0)),
        pl.BlockSpec((1, D), lambda i: (0, 0)),
    ]
    args += [wt, wb, b]
    out_specs = [pl.BlockSpec((_BN, D), lambda i: (i, 0))]
    out_shape = [jax.ShapeDtypeStruct((NP, D), jnp.float32)]
    if relu_and_next:
        out_specs.append(pl.BlockSpec((_BN, D), lambda i: (i, 0)))
        out_shape.append(jax.ShapeDtypeStruct((NP, D), jnp.float32))

    return pl.pallas_call(
        body,
        grid=(NP // _BN,),
        in_specs=in_specs,
        out_specs=out_specs if relu_and_next else out_specs[0],
        out_shape=out_shape if relu_and_next else out_shape[0],
    )(*args)


def _tc_link(g, w1, b1, w2, b2):
    def body(hs_ref, hr_ref, w1_ref, b1_ref, w2_ref, b2_ref, o_ref):
        x = hs_ref[...] * hr_ref[...]
        y = jnp.maximum(
            jnp.dot(x, w1_ref[...], preferred_element_type=jnp.float32)
            + b1_ref[...], 0.0)
        o_ref[...] = (jnp.dot(y, w2_ref[...],
                              preferred_element_type=jnp.float32)
                      + b2_ref[...])

    nblk = 2 * P // _BN
    return pl.pallas_call(
        body,
        grid=(nblk,),
        in_specs=[
            pl.BlockSpec((_BN, D), lambda i: (i, 0)),
            pl.BlockSpec((_BN, D), lambda i, n=nblk: (i + n, 0)),
            pl.BlockSpec((D, D), lambda i: (0, 0)),
            pl.BlockSpec((1, D), lambda i: (0, 0)),
            pl.BlockSpec((D, 1), lambda i: (0, 0)),
            pl.BlockSpec((1, 1), lambda i: (0, 0)),
        ],
        out_specs=pl.BlockSpec((_BN, 1), lambda i: (i, 0)),
        out_shape=jax.ShapeDtypeStruct((2 * P, 1), jnp.float32),
    )(g, g, w1, b1, w2, b2)


def kernel(gid, senders, receivers, pos_pairs, neg_pairs, embed,
           W1, b1, W2, b2, lW1, lb1, lW2, lb2):
    del gid
    cnt = _sc_degrees(senders, receivers).reshape(NC, 2, NP, 1)
    ss, rs, xs1 = _tc_scales(cnt, embed)

    acc1 = _sc_segsum(xs1, senders, receivers)
    h1, xs2 = _tc_layer(embed, acc1, xs1, rs, ss, W1[:D], W1[D:],
                        b1.reshape(1, D), relu_and_next=True)
    acc2 = _sc_segsum(xs2, senders, receivers)
    h = _tc_layer(h1, acc2, xs2, rs, None, W2[:D], W2[D:],
                  b2.reshape(1, D), relu_and_next=False)

    idx_all = jnp.concatenate([pos_pairs[:, 0], neg_pairs[:, 0],
                               pos_pairs[:, 1], neg_pairs[:, 1]])
    g = _sc_pair_gather(h, idx_all.reshape(NW, PPW))
    z = _tc_link(g, lW1, lb1.reshape(1, D),
                 lW2, lb2.reshape(1, 1))[:, 0]
    return z[:P], z[P:2 * P]


# speedup vs baseline: 1.0277x; 1.0277x over previous
import functools

import jax
import jax.numpy as jnp
from jax import lax
from jax.experimental import pallas as pl
from jax.experimental.pallas import tpu as pltpu
from jax.experimental.pallas import tpu_sc as plsc

N = 10000
E = 320000
D = 128
P = 8192
D2 = D // 2

NC = 2
NS = 16
NW = NC * NS

NP = 10240
RPT = NP // NS
HR = NP // 16
RPT16 = HR // NS
IDC = HR // 128

EPW = E // NW
K = 80
NCH = EPW // K

PPW = 4 * P // NW
PK = 128
PCH = PPW // PK

_mesh = plsc.VectorSubcoreMesh(core_axis_name="c", subcore_axis_name="s")


def _wid():
    return lax.axis_index("s") * NC + lax.axis_index("c")


HB = NP // 128
HBT = 8


@functools.partial(
    pl.kernel,
    out_type=jax.ShapeDtypeStruct((NC, 2, HB, 128), jnp.float32),
    mesh=_mesh,
    compiler_params=pltpu.CompilerParams(needs_layout_passes=False,
                                         use_tc_tiling_on_sc=False),
    scratch_types=[
        pltpu.VMEM((EPW,), jnp.int32),
        pltpu.VMEM((EPW,), jnp.int32),
        pltpu.VMEM((NP,), jnp.float32),
        pltpu.VMEM((NP,), jnp.float32),
        pltpu.VMEM((HB, 128), jnp.float32),
        pltpu.VMEM((HB, 128), jnp.float32),
        pltpu.VMEM((HB,), jnp.int32),
        pltpu.VMEM((HBT, 128), jnp.float32),
        pltpu.VMEM_SHARED((HB, 128), jnp.float32),
        pltpu.VMEM_SHARED((HB, 128), jnp.float32),
    ],
)
def _sc_degrees(s_hbm, r_hbm, out_hbm, sidx_v, ridx_v, hist_s1, hist_r1,
                hist_s, hist_r, idv, zb, cnt_s_sh, cnt_r_sh):
    cid = lax.axis_index("c")
    sid = lax.axis_index("s")
    base = _wid() * EPW
    pltpu.sync_copy(s_hbm.at[pl.ds(base, EPW)], sidx_v)
    pltpu.sync_copy(r_hbm.at[pl.ds(base, EPW)], ridx_v)

    zeros16 = jnp.zeros((16,), jnp.float32)
    ones16 = jnp.ones((16,), jnp.float32)
    iota16 = lax.iota(jnp.int32, 16)

    def _zero_hist(i, _):
        hist_s1[pl.ds(i * 16, 16)] = zeros16
        hist_r1[pl.ds(i * 16, 16)] = zeros16
        return 0

    lax.fori_loop(0, NP // 16, _zero_hist, 0)
    for i in range(HBT):
        for j in range(8):
            zb[i, pl.ds(j * 16, 16)] = zeros16
    for g in range(HB // 16):
        idv[pl.ds(g * 16, 16)] = g * 16 + iota16

    @pl.when(sid < HB // HBT)
    def _zero_counts():
        pltpu.sync_copy(zb, cnt_s_sh.at[pl.ds(sid * HBT, HBT)])
        pltpu.sync_copy(zb, cnt_r_sh.at[pl.ds(sid * HBT, HBT)])
    plsc.subcore_barrier()

    def _hist(i, _):
        sv = sidx_v[pl.ds(i * 16, 16)]
        plsc.addupdate_scatter(hist_s1, [sv], ones16)
        rv = ridx_v[pl.ds(i * 16, 16)]
        plsc.addupdate_scatter(hist_r1, [rv], ones16)
        return 0

    lax.fori_loop(0, EPW // 16, _hist, 0)

    def _pack(i, _):
        for j in range(8):
            hist_s[i, pl.ds(j * 16, 16)] = hist_s1[pl.ds(i * 128 + j * 16, 16)]
            hist_r[i, pl.ds(j * 16, 16)] = hist_r1[pl.ds(i * 128 + j * 16, 16)]
        return 0

    lax.fori_loop(0, HB, _pack, 0)

    pltpu.sync_copy(hist_s, cnt_s_sh.at[idv], add=True)
    pltpu.sync_copy(hist_r, cnt_r_sh.at[idv], add=True)
    plsc.subcore_barrier()

    @pl.when(sid < HB // HBT)
    def _write_counts():
        pltpu.sync_copy(cnt_s_sh.at[pl.ds(sid * HBT, HBT)],
                        out_hbm.at[cid, 0, pl.ds(sid * HBT, HBT)])
        pltpu.sync_copy(cnt_r_sh.at[pl.ds(sid * HBT, HBT)],
                        out_hbm.at[cid, 1, pl.ds(sid * HBT, HBT)])


@functools.partial(
    pl.kernel,
    out_type=jax.ShapeDtypeStruct((NC, NP, D), jnp.float32),
    mesh=_mesh,
    compiler_params=pltpu.CompilerParams(needs_layout_passes=False,
                                         use_tc_tiling_on_sc=True),
    scratch_types=[
        pltpu.VMEM((K,), jnp.int32), pltpu.VMEM((K,), jnp.int32),
        pltpu.VMEM((K,), jnp.int32), pltpu.VMEM((K,), jnp.int32),
        pltpu.VMEM((2, K, D), jnp.float32),
        pltpu.VMEM_SHARED((NP, D), jnp.float32),
        pltpu.SemaphoreType.DMA, pltpu.SemaphoreType.DMA,
        pltpu.SemaphoreType.DMA, pltpu.SemaphoreType.DMA,
    ],
)
def _sc_segsum(xs_hbm, sidx_hbm, ridx_hbm, out_hbm,
               sc0, sc1, rc0, rc1, rows_v, acc_sh, si0, si1, sg0, sg1):
    cid = lax.axis_index("c")
    sid = lax.axis_index("s")
    base = _wid() * EPW

    scurs = (sc0, sc1)
    rcurs = (rc0, rc1)
    sems_i = (si0, si1)
    sems_g = (sg0, sg1)

    zeros16 = jnp.zeros((16,), jnp.float32)

    def _zero_row(i, _):
        for j in range(D // 16):
            rows_v[0, i, pl.ds(j * 16, 16)] = zeros16
        return 0

    lax.fori_loop(0, K, _zero_row, 0)
    for r in range(RPT // K):
        pltpu.async_copy(rows_v.at[0],
                         acc_sh.at[pl.ds(sid * RPT + r * K, K)], si0)
    for r in range(RPT // K):
        pltpu.make_async_copy(rows_v.at[0],
                              acc_sh.at[pl.ds(sid * RPT + r * K, K)],
                              si0).wait()
    plsc.subcore_barrier()

    def _fire_idx(cc, j):
        pltpu.async_copy(sidx_hbm.at[pl.ds(base + cc * K, K)], scurs[j],
                         sems_i[j])
        pltpu.async_copy(ridx_hbm.at[pl.ds(base + cc * K, K)], rcurs[j],
                         sems_i[j])

    def _wait_idx(cc, j):
        pltpu.make_async_copy(sidx_hbm.at[pl.ds(base + cc * K, K)], scurs[j],
                              sems_i[j]).wait()
        pltpu.make_async_copy(ridx_hbm.at[pl.ds(base + cc * K, K)], rcurs[j],
                              sems_i[j]).wait()

    def _fire_g(j, b):
        pltpu.async_copy(xs_hbm.at[scurs[j]], rows_v.at[b], sems_g[b])

    def _drain_scatter(j, b):
        pltpu.make_async_copy(xs_hbm.at[scurs[j]], rows_v.at[b],
                              sems_g[b]).wait()
        pltpu.sync_copy(rows_v.at[b], acc_sh.at[rcurs[j]], add=True)

    _fire_idx(0, 0)
    _fire_idx(1, 1)
    _wait_idx(0, 0)
    _fire_g(0, 0)

    def _body(hc, _):
        c0 = hc * 2
        _wait_idx(c0 + 1, 1)
        _fire_g(1, 1)
        _drain_scatter(0, 0)
        _fire_idx(c0 + 2, 0)
        _drain_scatter(1, 1)

        @pl.when(c0 + 3 < NCH)
        def _():
            _fire_idx(c0 + 3, 1)
        _wait_idx(c0 + 2, 0)
        _fire_g(0, 0)
        return 0

    lax.fori_loop(0, (NCH - 1) // 2, _body, 0)
    _drain_scatter(0, 0)

    plsc.subcore_barrier()
    semo = (si0, si1)
    for r in range(RPT // K):
        b = r % 2
        if r >= 2:
            pltpu.make_async_copy(
                rows_v.at[b],
                out_hbm.at[cid].at[pl.ds(sid * RPT + (r - 2) * K, K)],
                semo[b]).wait()
        pltpu.sync_copy(acc_sh.at[pl.ds(sid * RPT + r * K, K)], rows_v.at[b])
        pltpu.async_copy(rows_v.at[b],
                         out_hbm.at[cid].at[pl.ds(sid * RPT + r * K, K)],
                         semo[b])
    for r in range(RPT // K - 2, RPT // K):
        b = r % 2
        pltpu.make_async_copy(
            rows_v.at[b],
            out_hbm.at[cid].at[pl.ds(sid * RPT + r * K, K)], semo[b]).wait()


@functools.partial(
    pl.kernel,
    out_type=jax.ShapeDtypeStruct((4 * P, D), jnp.float32),
    mesh=_mesh,
    compiler_params=pltpu.CompilerParams(needs_layout_passes=False,
                                         use_tc_tiling_on_sc=True),
    scratch_types=[
        pltpu.VMEM((PPW,), jnp.int32),
        pltpu.VMEM((2, PK, D), jnp.float32),
        pltpu.SemaphoreType.DMA,
        pltpu.SemaphoreType.DMA,
    ],
)
def _sc_pair_gather(h_hbm, idx_hbm, out_hbm, idx_v, rows_v, sem0, sem1):
    wid = _wid()
    pltpu.sync_copy(idx_hbm.at[wid], idx_v)
    sems = (sem0, sem1)

    def _fire(c, b):
        pltpu.async_copy(h_hbm.at[idx_v.at[pl.ds(c * PK, PK)]], rows_v.at[b],
                         sems[b])

    def _drain_write(c, b):
        pltpu.make_async_copy(h_hbm.at[idx_v.at[pl.ds(c * PK, PK)]],
                              rows_v.at[b], sems[b]).wait()
        pltpu.sync_copy(rows_v.at[b],
                        out_hbm.at[pl.ds(wid * PPW + c * PK, PK)])

    _fire(0, 0)
    for c in range(PCH):
        if c + 1 < PCH:
            _fire(c + 1, (c + 1) % 2)
        _drain_write(c, c % 2)


_BN = 2560


def _tc_scales(cnt, embed_p):
    def body(cnt_ref, emb_ref, ss_ref, rs_ref, xs_ref):
        cs = cnt_ref[0, 0] + cnt_ref[1, 0] + 1.0
        cr = cnt_ref[0, 1] + cnt_ref[1, 1] + 1.0
        ss = lax.rsqrt(cs)
        ss_ref[...] = ss
        rs_ref[...] = lax.rsqrt(cr) / cr
        xs_ref[...] = emb_ref[...] * ss

    return pl.pallas_call(
        body,
        grid=(NP // _BN,),
        in_specs=[
            pl.BlockSpec((NC, 2, _BN, 1), lambda i: (0, 0, i, 0)),
            pl.BlockSpec((_BN, D), lambda i: (i, 0)),
        ],
        out_specs=[
            pl.BlockSpec((_BN, 1), lambda i: (i, 0)),
            pl.BlockSpec((_BN, 1), lambda i: (i, 0)),
            pl.BlockSpec((_BN, D), lambda i: (i, 0)),
        ],
        out_shape=[
            jax.ShapeDtypeStruct((NP, 1), jnp.float32),
            jax.ShapeDtypeStruct((NP, 1), jnp.float32),
            jax.ShapeDtypeStruct((NP, D), jnp.float32),
        ],
    )(cnt, embed_p)


def _tc_layer(x, acc, xs, rs, ss, wt, wb, b, *, relu_and_next):
    def body(*refs):
        if relu_and_next:
            (x_ref, acc_ref, xs_ref, rs_ref, ss_ref, wt_ref, wb_ref, b_ref,
             h_ref, xs2_ref) = refs
        else:
            (x_ref, acc_ref, xs_ref, rs_ref, wt_ref, wb_ref, b_ref,
             h_ref) = refs
        m = (acc_ref[0] + acc_ref[1] + xs_ref[...]) * rs_ref[...]
        h = (jnp.dot(x_ref[...], wt_ref[...],
                     preferred_element_type=jnp.float32)
             + jnp.dot(m, wb_ref[...], preferred_element_type=jnp.float32)
             + b_ref[...])
        if relu_and_next:
            h = jnp.maximum(h, 0.0)
            h_ref[...] = h
            xs2_ref[...] = h * ss_ref[...]
        else:
            h_ref[...] = h

    in_specs = [
        pl.BlockSpec((_BN, D), lambda i: (i, 0)),
        pl.BlockSpec((NC, _BN, D), lambda i: (0, i, 0)),
        pl.BlockSpec((_BN, D), lambda i: (i, 0)),
        pl.BlockSpec((_BN, 1), lambda i: (i, 0)),
    ]
    args = [x, acc, xs, rs]
    if relu_and_next:
        in_specs.append(pl.BlockSpec((_BN, 1), lambda i: (i, 0)))
        args.append(ss)
    in_specs += [
        pl.BlockSpec((D, D), lambda i: (0, 0)),
        pl.BlockSpec((D, D), lambda i: (0, 0)),
        pl.BlockSpec((1, D), lambda i: (0, 0)),
    ]
    args += [wt, wb, b]
    out_specs = [pl.BlockSpec((_BN, D), lambda i: (i, 0))]
    out_shape = [jax.ShapeDtypeStruct((NP, D), jnp.float32)]
    if relu_and_next:
        out_specs.append(pl.BlockSpec((_BN, D), lambda i: (i, 0)))
        out_shape.append(jax.ShapeDtypeStruct((NP, D), jnp.float32))

    return pl.pallas_call(
        body,
        grid=(NP // _BN,),
        in_specs=in_specs,
        out_specs=out_specs if relu_and_next else out_specs[0],
        out_shape=out_shape if relu_and_next else out_shape[0],
    )(*args)


def _tc_link(g, w1, b1, w2, b2):
    def body(hs_ref, hr_ref, w1_ref, b1_ref, w2_ref, b2_ref, o_ref):
        x = hs_ref[...] * hr_ref[...]
        y = jnp.maximum(
            jnp.dot(x, w1_ref[...], preferred_element_type=jnp.float32)
            + b1_ref[...], 0.0)
        o_ref[...] = (jnp.dot(y, w2_ref[...],
                              preferred_element_type=jnp.float32)
                      + b2_ref[...])

    nblk = 2 * P // _BN
    return pl.pallas_call(
        body,
        grid=(nblk,),
        in_specs=[
            pl.BlockSpec((_BN, D), lambda i: (i, 0)),
            pl.BlockSpec((_BN, D), lambda i, n=nblk: (i + n, 0)),
            pl.BlockSpec((D, D), lambda i: (0, 0)),
            pl.BlockSpec((1, D), lambda i: (0, 0)),
            pl.BlockSpec((D, 1), lambda i: (0, 0)),
            pl.BlockSpec((1, 1), lambda i: (0, 0)),
        ],
        out_specs=pl.BlockSpec((_BN, 1), lambda i: (i, 0)),
        out_shape=jax.ShapeDtypeStruct((2 * P, 1), jnp.float32),
    )(g, g, w1, b1, w2, b2)


def kernel(gid, senders, receivers, pos_pairs, neg_pairs, embed,
           W1, b1, W2, b2, lW1, lb1, lW2, lb2):
    del gid
    cnt = _sc_degrees(senders, receivers).reshape(NC, 2, NP, 1)
    ss, rs, xs1 = _tc_scales(cnt, embed)

    acc1 = _sc_segsum(xs1, senders, receivers)
    h1, xs2 = _tc_layer(embed, acc1, xs1, rs, ss, W1[:D], W1[D:],
                        b1.reshape(1, D), relu_and_next=True)
    acc2 = _sc_segsum(xs2, senders, receivers)
    h = _tc_layer(h1, acc2, xs2, rs, None, W2[:D], W2[D:],
                  b2.reshape(1, D), relu_and_next=False)

    idx_all = jnp.concatenate([pos_pairs[:, 0], neg_pairs[:, 0],
                               pos_pairs[:, 1], neg_pairs[:, 1]])
    g = _sc_pair_gather(h, idx_all.reshape(NW, PPW))
    z = _tc_link(g, lW1, lb1.reshape(1, D),
                 lW2, lb2.reshape(1, 1))[:, 0]
    return z[:P], z[P:2 * P]


# speedup vs baseline: 1.0283x; 1.0006x over previous
import functools

import jax
import jax.numpy as jnp
from jax import lax
from jax.experimental import pallas as pl
from jax.experimental.pallas import tpu as pltpu
from jax.experimental.pallas import tpu_sc as plsc

N = 10000
E = 320000
D = 128
P = 8192
D2 = D // 2

NC = 2
NS = 16
NW = NC * NS

NP = 10240
RPT = NP // NS
HR = NP // 16
RPT16 = HR // NS
IDC = HR // 128

EPW = E // NW
K = 80
NCH = EPW // K

PPW = 4 * P // NW
PK = 128
PCH = PPW // PK

_mesh = plsc.VectorSubcoreMesh(core_axis_name="c", subcore_axis_name="s")


def _wid():
    return lax.axis_index("s") * NC + lax.axis_index("c")


HB = NP // 128
HBT = 8


@functools.partial(
    pl.kernel,
    out_type=jax.ShapeDtypeStruct((NC, 2, HB, 128), jnp.float32),
    mesh=_mesh,
    compiler_params=pltpu.CompilerParams(needs_layout_passes=False,
                                         use_tc_tiling_on_sc=False),
    scratch_types=[
        pltpu.VMEM((EPW,), jnp.int32),
        pltpu.VMEM((EPW,), jnp.int32),
        pltpu.VMEM((NP,), jnp.float32),
        pltpu.VMEM((NP,), jnp.float32),
        pltpu.VMEM((HB, 128), jnp.float32),
        pltpu.VMEM((HB, 128), jnp.float32),
        pltpu.VMEM((HB,), jnp.int32),
        pltpu.VMEM((HBT, 128), jnp.float32),
        pltpu.VMEM_SHARED((HB, 128), jnp.float32),
        pltpu.VMEM_SHARED((HB, 128), jnp.float32),
    ],
)
def _sc_degrees(s_hbm, r_hbm, out_hbm, sidx_v, ridx_v, hist_s1, hist_r1,
                hist_s, hist_r, idv, zb, cnt_s_sh, cnt_r_sh):
    cid = lax.axis_index("c")
    sid = lax.axis_index("s")
    base = _wid() * EPW
    pltpu.sync_copy(s_hbm.at[pl.ds(base, EPW)], sidx_v)
    pltpu.sync_copy(r_hbm.at[pl.ds(base, EPW)], ridx_v)

    zeros16 = jnp.zeros((16,), jnp.float32)
    ones16 = jnp.ones((16,), jnp.float32)
    iota16 = lax.iota(jnp.int32, 16)

    def _zero_hist(i, _):
        hist_s1[pl.ds(i * 16, 16)] = zeros16
        hist_r1[pl.ds(i * 16, 16)] = zeros16
        return 0

    lax.fori_loop(0, NP // 16, _zero_hist, 0)
    for i in range(HBT):
        for j in range(8):
            zb[i, pl.ds(j * 16, 16)] = zeros16
    for g in range(HB // 16):
        idv[pl.ds(g * 16, 16)] = g * 16 + iota16

    @pl.when(sid < HB // HBT)
    def _zero_counts():
        pltpu.sync_copy(zb, cnt_s_sh.at[pl.ds(sid * HBT, HBT)])
        pltpu.sync_copy(zb, cnt_r_sh.at[pl.ds(sid * HBT, HBT)])
    plsc.subcore_barrier()

    def _hist(i, _):
        sv = sidx_v[pl.ds(i * 16, 16)]
        plsc.addupdate_scatter(hist_s1, [sv], ones16)
        rv = ridx_v[pl.ds(i * 16, 16)]
        plsc.addupdate_scatter(hist_r1, [rv], ones16)
        return 0

    lax.fori_loop(0, EPW // 16, _hist, 0)

    def _pack(i, _):
        for j in range(8):
            hist_s[i, pl.ds(j * 16, 16)] = hist_s1[pl.ds(i * 128 + j * 16, 16)]
            hist_r[i, pl.ds(j * 16, 16)] = hist_r1[pl.ds(i * 128 + j * 16, 16)]
        return 0

    lax.fori_loop(0, HB, _pack, 0)

    pltpu.sync_copy(hist_s, cnt_s_sh.at[idv], add=True)
    pltpu.sync_copy(hist_r, cnt_r_sh.at[idv], add=True)
    plsc.subcore_barrier()

    @pl.when(sid < HB // HBT)
    def _write_counts():
        pltpu.sync_copy(cnt_s_sh.at[pl.ds(sid * HBT, HBT)],
                        out_hbm.at[cid, 0, pl.ds(sid * HBT, HBT)])
        pltpu.sync_copy(cnt_r_sh.at[pl.ds(sid * HBT, HBT)],
                        out_hbm.at[cid, 1, pl.ds(sid * HBT, HBT)])


@functools.partial(
    pl.kernel,
    out_type=jax.ShapeDtypeStruct((NC, NP, D), jnp.float32),
    mesh=_mesh,
    compiler_params=pltpu.CompilerParams(needs_layout_passes=False,
                                         use_tc_tiling_on_sc=True),
    scratch_types=[
        pltpu.VMEM((K,), jnp.int32), pltpu.VMEM((K,), jnp.int32),
        pltpu.VMEM((K,), jnp.int32), pltpu.VMEM((K,), jnp.int32),
        pltpu.VMEM((2, K, D), jnp.float32),
        pltpu.VMEM_SHARED((NP, D), jnp.float32),
        pltpu.SemaphoreType.DMA, pltpu.SemaphoreType.DMA,
        pltpu.SemaphoreType.DMA, pltpu.SemaphoreType.DMA,
    ],
)
def _sc_segsum(xs_hbm, sidx_hbm, ridx_hbm, out_hbm,
               sc0, sc1, rc0, rc1, rows_v, acc_sh, si0, si1, sg0, sg1):
    cid = lax.axis_index("c")
    sid = lax.axis_index("s")
    base = _wid() * EPW

    scurs = (sc0, sc1)
    rcurs = (rc0, rc1)
    sems_i = (si0, si1)
    sems_g = (sg0, sg1)

    zeros16 = jnp.zeros((16,), jnp.float32)

    def _zero_row(i, _):
        for j in range(D // 16):
            rows_v[0, i, pl.ds(j * 16, 16)] = zeros16
        return 0

    lax.fori_loop(0, K, _zero_row, 0)
    for r in range(RPT // K):
        pltpu.async_copy(rows_v.at[0],
                         acc_sh.at[pl.ds(sid * RPT + r * K, K)], si0)
    for r in range(RPT // K):
        pltpu.make_async_copy(rows_v.at[0],
                              acc_sh.at[pl.ds(sid * RPT + r * K, K)],
                              si0).wait()
    plsc.subcore_barrier()

    def _fire_idx(cc, j):
        pltpu.async_copy(sidx_hbm.at[pl.ds(base + cc * K, K)], scurs[j],
                         sems_i[j])
        pltpu.async_copy(ridx_hbm.at[pl.ds(base + cc * K, K)], rcurs[j],
                         sems_i[j])

    def _wait_idx(cc, j):
        pltpu.make_async_copy(sidx_hbm.at[pl.ds(base + cc * K, K)], scurs[j],
                              sems_i[j]).wait()
        pltpu.make_async_copy(ridx_hbm.at[pl.ds(base + cc * K, K)], rcurs[j],
                              sems_i[j]).wait()

    def _fire_g(j, b):
        pltpu.async_copy(xs_hbm.at[scurs[j]], rows_v.at[b], sems_g[b])

    def _drain_scatter(j, b):
        pltpu.make_async_copy(xs_hbm.at[scurs[j]], rows_v.at[b],
                              sems_g[b]).wait()
        pltpu.sync_copy(rows_v.at[b], acc_sh.at[rcurs[j]], add=True)

    _fire_idx(0, 0)
    _fire_idx(1, 1)
    _wait_idx(0, 0)
    _fire_g(0, 0)

    def _body(hc, _):
        c0 = hc * 2
        _wait_idx(c0 + 1, 1)
        _fire_g(1, 1)
        _drain_scatter(0, 0)
        _fire_idx(c0 + 2, 0)
        _drain_scatter(1, 1)

        @pl.when(c0 + 3 < NCH)
        def _():
            _fire_idx(c0 + 3, 1)
        _wait_idx(c0 + 2, 0)
        _fire_g(0, 0)
        return 0

    lax.fori_loop(0, (NCH - 1) // 2, _body, 0)
    _drain_scatter(0, 0)

    plsc.subcore_barrier()
    semo = (si0, si1)
    for r in range(RPT // K):
        b = r % 2
        if r >= 2:
            pltpu.make_async_copy(
                rows_v.at[b],
                out_hbm.at[cid].at[pl.ds(sid * RPT + (r - 2) * K, K)],
                semo[b]).wait()
        pltpu.sync_copy(acc_sh.at[pl.ds(sid * RPT + r * K, K)], rows_v.at[b])
        pltpu.async_copy(rows_v.at[b],
                         out_hbm.at[cid].at[pl.ds(sid * RPT + r * K, K)],
                         semo[b])
    for r in range(RPT // K - 2, RPT // K):
        b = r % 2
        pltpu.make_async_copy(
            rows_v.at[b],
            out_hbm.at[cid].at[pl.ds(sid * RPT + r * K, K)], semo[b]).wait()


@functools.partial(
    pl.kernel,
    out_type=jax.ShapeDtypeStruct((4 * P, D), jnp.float32),
    mesh=_mesh,
    compiler_params=pltpu.CompilerParams(needs_layout_passes=False,
                                         use_tc_tiling_on_sc=True),
    scratch_types=[
        pltpu.VMEM((PPW,), jnp.int32),
        pltpu.VMEM((2, PK, D), jnp.float32),
        pltpu.SemaphoreType.DMA,
        pltpu.SemaphoreType.DMA,
    ],
)
def _sc_pair_gather(h_hbm, idx_hbm, out_hbm, idx_v, rows_v, sem0, sem1):
    wid = _wid()
    pltpu.sync_copy(idx_hbm.at[wid], idx_v)
    sems = (sem0, sem1)

    def _fire(c, b):
        pltpu.async_copy(h_hbm.at[idx_v.at[pl.ds(c * PK, PK)]], rows_v.at[b],
                         sems[b])

    def _drain_write(c, b):
        pltpu.make_async_copy(h_hbm.at[idx_v.at[pl.ds(c * PK, PK)]],
                              rows_v.at[b], sems[b]).wait()
        pltpu.sync_copy(rows_v.at[b],
                        out_hbm.at[pl.ds(wid * PPW + c * PK, PK)])

    _fire(0, 0)
    for c in range(PCH):
        if c + 1 < PCH:
            _fire(c + 1, (c + 1) % 2)
        _drain_write(c, c % 2)


_BN = 5120


def _tc_scales(cnt, embed_p):
    def body(cnt_ref, emb_ref, ss_ref, rs_ref, xs_ref):
        cs = cnt_ref[0, 0] + cnt_ref[1, 0] + 1.0
        cr = cnt_ref[0, 1] + cnt_ref[1, 1] + 1.0
        ss = lax.rsqrt(cs)
        ss_ref[...] = ss
        rs_ref[...] = lax.rsqrt(cr) / cr
        xs_ref[...] = emb_ref[...] * ss

    return pl.pallas_call(
        body,
        grid=(NP // _BN,),
        in_specs=[
            pl.BlockSpec((NC, 2, _BN, 1), lambda i: (0, 0, i, 0)),
            pl.BlockSpec((_BN, D), lambda i: (i, 0)),
        ],
        out_specs=[
            pl.BlockSpec((_BN, 1), lambda i: (i, 0)),
            pl.BlockSpec((_BN, 1), lambda i: (i, 0)),
            pl.BlockSpec((_BN, D), lambda i: (i, 0)),
        ],
        out_shape=[
            jax.ShapeDtypeStruct((NP, 1), jnp.float32),
            jax.ShapeDtypeStruct((NP, 1), jnp.float32),
            jax.ShapeDtypeStruct((NP, D), jnp.float32),
        ],
    )(cnt, embed_p)


def _tc_layer(x, acc, xs, rs, ss, wt, wb, b, *, relu_and_next):
    def body(*refs):
        if relu_and_next:
            (x_ref, acc_ref, xs_ref, rs_ref, ss_ref, wt_ref, wb_ref, b_ref,
             h_ref, xs2_ref) = refs
        else:
            (x_ref, acc_ref, xs_ref, rs_ref, wt_ref, wb_ref, b_ref,
             h_ref) = refs
        m = (acc_ref[0] + acc_ref[1] + xs_ref[...]) * rs_ref[...]
        h = (jnp.dot(x_ref[...], wt_ref[...],
                     preferred_element_type=jnp.float32)
             + jnp.dot(m, wb_ref[...], preferred_element_type=jnp.float32)
             + b_ref[...])
        if relu_and_next:
            h = jnp.maximum(h, 0.0)
            h_ref[...] = h
            xs2_ref[...] = h * ss_ref[...]
        else:
            h_ref[...] = h

    in_specs = [
        pl.BlockSpec((_BN, D), lambda i: (i, 0)),
        pl.BlockSpec((NC, _BN, D), lambda i: (0, i, 0)),
        pl.BlockSpec((_BN, D), lambda i: (i, 0)),
        pl.BlockSpec((_BN, 1), lambda i: (i, 0)),
    ]
    args = [x, acc, xs, rs]
    if relu_and_next:
        in_specs.append(pl.BlockSpec((_BN, 1), lambda i: (i, 0)))
        args.append(ss)
    in_specs += [
        pl.BlockSpec((D, D), lambda i: (0, 0)),
        pl.BlockSpec((D, D), lambda i: (0, 0)),
        pl.BlockSpec((1, D), lambda i: (0, 0)),
    ]
    args += [wt, wb, b]
    out_specs = [pl.BlockSpec((_BN, D), lambda i: (i, 0))]
    out_shape = [jax.ShapeDtypeStruct((NP, D), jnp.float32)]
    if relu_and_next:
        out_specs.append(pl.BlockSpec((_BN, D), lambda i: (i, 0)))
        out_shape.append(jax.ShapeDtypeStruct((NP, D), jnp.float32))

    return pl.pallas_call(
        body,
        grid=(NP // _BN,),
        in_specs=in_specs,
        out_specs=out_specs if relu_and_next else out_specs[0],
        out_shape=out_shape if relu_and_next else out_shape[0],
    )(*args)


def _tc_link(g, w1, b1, w2, b2):
    def body(hs_ref, hr_ref, w1_ref, b1_ref, w2_ref, b2_ref, o_ref):
        x = hs_ref[...] * hr_ref[...]
        y = jnp.maximum(
            jnp.dot(x, w1_ref[...], preferred_element_type=jnp.float32)
            + b1_ref[...], 0.0)
        o_ref[...] = (jnp.dot(y, w2_ref[...],
                              preferred_element_type=jnp.float32)
                      + b2_ref[...])

    nblk = 2 * P // _BN
    return pl.pallas_call(
        body,
        grid=(nblk,),
        in_specs=[
            pl.BlockSpec((_BN, D), lambda i: (i, 0)),
            pl.BlockSpec((_BN, D), lambda i, n=nblk: (i + n, 0)),
            pl.BlockSpec((D, D), lambda i: (0, 0)),
            pl.BlockSpec((1, D), lambda i: (0, 0)),
            pl.BlockSpec((D, 1), lambda i: (0, 0)),
            pl.BlockSpec((1, 1), lambda i: (0, 0)),
        ],
        out_specs=pl.BlockSpec((_BN, 1), lambda i: (i, 0)),
        out_shape=jax.ShapeDtypeStruct((2 * P, 1), jnp.float32),
    )(g, g, w1, b1, w2, b2)


def kernel(gid, senders, receivers, pos_pairs, neg_pairs, embed,
           W1, b1, W2, b2, lW1, lb1, lW2, lb2):
    del gid
    cnt = _sc_degrees(senders, receivers).reshape(NC, 2, NP, 1)
    ss, rs, xs1 = _tc_scales(cnt, embed)

    acc1 = _sc_segsum(xs1, senders, receivers)
    h1, xs2 = _tc_layer(embed, acc1, xs1, rs, ss, W1[:D], W1[D:],
                        b1.reshape(1, D), relu_and_next=True)
    acc2 = _sc_segsum(xs2, senders, receivers)
    h = _tc_layer(h1, acc2, xs2, rs, None, W2[:D], W2[D:],
                  b2.reshape(1, D), relu_and_next=False)

    idx_all = jnp.concatenate([pos_pairs[:, 0], neg_pairs[:, 0],
                               pos_pairs[:, 1], neg_pairs[:, 1]])
    g = _sc_pair_gather(h, idx_all.reshape(NW, PPW))
    z = _tc_link(g, lW1, lb1.reshape(1, D),
                 lW2, lb2.reshape(1, 1))[:, 0]
    return z[:P], z[P:2 * P]
